# baseline (speedup 1.0000x reference)
import functools

import jax
import jax.numpy as jnp
import numpy as np
from jax import lax
from jax.experimental import pallas as pl
from jax.experimental.pallas import tpu as pltpu
from jax.experimental.pallas import tpu_sc as plsc

B = 8
NPB = 4096
D = 64
S = 2048
K = 64
R2 = np.float32(0.2 * 0.2)
DOUT = 128

NW = 32
CPW = (B * S) // NW
IDXBUF = 80
BLK = 64


def _fps_body(px_ref, py_ref, pz_ref, sel_ref, cx_ref, cy_ref, cz_ref):
    px = px_ref[...]
    py = py_ref[...]
    pz = pz_ref[...]
    i32 = jnp.int32
    r = lax.broadcasted_iota(i32, (B, 32, 128), 1)
    l = lax.broadcasted_iota(i32, (B, 32, 128), 2)
    pt_iota = r * 128 + l
    rs = lax.broadcasted_iota(i32, (B, 16, 128), 1)
    ls = lax.broadcasted_iota(i32, (B, 16, 128), 2)
    sel_iota = rs * 128 + ls

    d0 = jnp.full((B, 32, 128), jnp.inf, jnp.float32)
    arg0 = jnp.zeros((B, 1, 1), i32)
    sel0 = jnp.zeros((B, 16, 128), i32)
    c0 = jnp.zeros((B, 16, 128), jnp.float32)

    def body(i, st):
        arg_prev, d, sel, cx, cy, cz = st
        onehot = pt_iota == arg_prev
        cxs = jnp.sum(jnp.where(onehot, px, 0.0), axis=(1, 2), keepdims=True)
        cys = jnp.sum(jnp.where(onehot, py, 0.0), axis=(1, 2), keepdims=True)
        czs = jnp.sum(jnp.where(onehot, pz, 0.0), axis=(1, 2), keepdims=True)
        wm = sel_iota == (i - 1)
        cx = jnp.where(wm, cxs, cx)
        cy = jnp.where(wm, cys, cy)
        cz = jnp.where(wm, czs, cz)
        dx = px - cxs
        dy = py - cys
        dz = pz - czs
        dd = dx * dx + dy * dy + dz * dz
        d = jnp.minimum(d, dd)
        maxd = jnp.max(d, axis=(1, 2), keepdims=True)
        cand = jnp.where(d == maxd, pt_iota, NPB)
        arg = jnp.min(cand, axis=(1, 2), keepdims=True)
        sel = jnp.where(sel_iota == i, arg, sel)
        return arg, d, sel, cx, cy, cz

    _, _, sel, cx, cy, cz = lax.fori_loop(
        1, S + 1, body, (arg0, d0, sel0, c0, c0, c0))
    sel_ref[...] = sel
    cx_ref[...] = cx
    cy_ref[...] = cy
    cz_ref[...] = cz


def _run_fps(px, py, pz):
    out_shapes = (
        jax.ShapeDtypeStruct((B, 16, 128), jnp.int32),
        jax.ShapeDtypeStruct((B, 16, 128), jnp.float32),
        jax.ShapeDtypeStruct((B, 16, 128), jnp.float32),
        jax.ShapeDtypeStruct((B, 16, 128), jnp.float32),
    )
    return pl.pallas_call(_fps_body, out_shape=out_shapes)(px, py, pz)


def _mm_body(a_ref, w_ref, o_ref):
    o_ref[...] = jnp.dot(a_ref[...], w_ref[...],
                         preferred_element_type=jnp.float32,
                         precision=lax.Precision.HIGHEST)


def _run_mm(a, w):
    n = a.shape[0]
    tile = 1024
    grid = (n // tile,)
    return pl.pallas_call(
        _mm_body,
        grid=grid,
        in_specs=[
            pl.BlockSpec((tile, 128), lambda i: (i, 0)),
            pl.BlockSpec((128, 128), lambda i: (0, 0)),
        ],
        out_specs=pl.BlockSpec((tile, 128), lambda i: (i, 0)),
        out_shape=jax.ShapeDtypeStruct((n, 128), jnp.float32),
    )(a, w)


def _sc_body(px_hbm, py_hbm, pz_hbm, sel_hbm, cx_hbm, cy_hbm, cz_hbm,
             q_hbm, z_hbm, out_hbm,
             px_v, py_v, pz_v, sel_v, cx_v, cy_v, cz_v,
             q_v, idx_v, zrows_v, outbuf_v, sem, gsem):
    i32 = jnp.int32
    f32 = jnp.float32
    wid = lax.axis_index("s") * 2 + lax.axis_index("c")
    cloud = wid // 4
    cbase = wid * CPW
    gbase = cloud * NPB

    pltpu.sync_copy(px_hbm.at[pl.ds(gbase, NPB)], px_v)
    pltpu.sync_copy(py_hbm.at[pl.ds(gbase, NPB)], py_v)
    pltpu.sync_copy(pz_hbm.at[pl.ds(gbase, NPB)], pz_v)
    pltpu.sync_copy(sel_hbm.at[pl.ds(cbase, CPW)], sel_v)
    pltpu.sync_copy(cx_hbm.at[pl.ds(cbase, CPW)], cx_v)
    pltpu.sync_copy(cy_hbm.at[pl.ds(cbase, CPW)], cy_v)
    pltpu.sync_copy(cz_hbm.at[pl.ds(cbase, CPW)], cz_v)

    lane = lax.broadcasted_iota(i32, (16,), 0)
    neg_inf = jnp.full((16,), -jnp.inf, f32)

    def _splat(v, t):
        tv = jnp.full((16,), t, i32)
        scalar = jnp.sum(jnp.where(lane == tv, v, jnp.zeros((16,), v.dtype)),
                         axis=0)
        return jnp.full((16,), scalar, v.dtype)

    def center_body(b, s, _):
        sw = b * BLK + s
        g = sw // 16
        t = sw - g * 16
        sel16 = sel_v[pl.ds(g * 16, 16)]
        cx16 = cx_v[pl.ds(g * 16, 16)]
        cy16 = cy_v[pl.ds(g * 16, 16)]
        cz16 = cz_v[pl.ds(g * 16, 16)]
        selv = _splat(sel16, t)
        cxv = _splat(cx16, t)
        cyv = _splat(cy16, t)
        czv = _splat(cz16, t)
        safe = selv + gbase
        for t in range(IDXBUF // 16):
            idx_v[pl.ds(t * 16, 16)] = safe

        def chunk_body(c, off):
            pxc = px_v[pl.ds(c * 16, 16)]
            pyc = py_v[pl.ds(c * 16, 16)]
            pzc = pz_v[pl.ds(c * 16, 16)]
            jidx = jnp.full((16,), c * 16, i32) + lane
            dx = pxc - cxv
            dy = pyc - cyv
            dz = pzc - czv
            d2 = dx * dx + dy * dy + dz * dz
            m = (d2 <= R2) & (jidx != selv)
            mi = m.astype(i32)
            incl = plsc.cumsum(mi)
            posn = off + incl - 1
            cnt = plsc.all_reduce_population_count(m)
            okm = m & (posn < IDXBUF)
            plsc.store_scatter(idx_v, [posn], jidx + gbase, mask=okm)
            return off + cnt

        off = lax.fori_loop(0, NPB // 16, chunk_body, jnp.zeros((16,), i32))
        nrows = jnp.max(jnp.minimum(off, jnp.full((16,), K, i32)))

        pltpu.async_copy(z_hbm.at[idx_v], zrows_v, gsem).wait()

        def row_body(rr, accs):
            return tuple(
                jnp.maximum(a, zrows_v[rr, pl.ds(kk * 16, 16)])
                for kk, a in enumerate(accs))

        accs = lax.fori_loop(0, nrows, row_body, (neg_inf,) * 8)
        for kk in range(8):
            res = jnp.maximum(accs[kk] - q_v[s, pl.ds(kk * 16, 16)], 0.0)
            outbuf_v[s, pl.ds(kk * 16, 16)] = res
        return 0

    def block_body(b, _):
        pltpu.sync_copy(q_hbm.at[pl.ds(cbase + b * BLK, BLK)], q_v)
        lax.fori_loop(0, BLK, lambda s, c: center_body(b, s, c), 0)
        pltpu.sync_copy(outbuf_v, out_hbm.at[pl.ds(cbase + b * BLK, BLK)])
        return 0

    lax.fori_loop(0, CPW // BLK, block_body, 0)


def _run_sc(px, py, pz, sel, cx, cy, cz, q, z):
    mesh = plsc.VectorSubcoreMesh(core_axis_name="c", subcore_axis_name="s")
    kfn = pl.kernel(
        _sc_body,
        out_type=jax.ShapeDtypeStruct((B * S, DOUT), jnp.float32),
        mesh=mesh,
        compiler_params=pltpu.CompilerParams(needs_layout_passes=False),
        scratch_types=[
            pltpu.VMEM((NPB,), jnp.float32),
            pltpu.VMEM((NPB,), jnp.float32),
            pltpu.VMEM((NPB,), jnp.float32),
            pltpu.VMEM((CPW,), jnp.int32),
            pltpu.VMEM((CPW,), jnp.float32),
            pltpu.VMEM((CPW,), jnp.float32),
            pltpu.VMEM((CPW,), jnp.float32),
            pltpu.VMEM((BLK, DOUT), jnp.float32),
            pltpu.VMEM((IDXBUF,), jnp.int32),
            pltpu.VMEM((IDXBUF, DOUT), jnp.float32),
            pltpu.VMEM((BLK, DOUT), jnp.float32),
            pltpu.SemaphoreType.DMA,
            pltpu.SemaphoreType.DMA,
        ],
    )
    return kfn(px, py, pz, sel, cx, cy, cz, q, z)


def kernel(x, pos, batch, W1, b1):
    posb = pos.reshape(B, NPB, 3)
    px = posb[:, :, 0].reshape(B, 32, 128)
    py = posb[:, :, 1].reshape(B, 32, 128)
    pz = posb[:, :, 2].reshape(B, 32, 128)

    sel3, cx3, cy3, cz3 = _run_fps(px, py, pz)
    sel = sel3.reshape(B * S)
    cx = cx3.reshape(B * S)
    cy = cy3.reshape(B * S)
    cz = cz3.reshape(B * S)
    pos_i = jnp.stack([cx, cy, cz], axis=-1)

    a_z = jnp.concatenate(
        [x, pos, jnp.zeros((B * NPB, 128 - D - 3), jnp.float32)], axis=1)
    a_q = jnp.concatenate(
        [jnp.zeros((B * S, D), jnp.float32), pos_i,
         jnp.full((B * S, 1), -1.0, jnp.float32),
         jnp.zeros((B * S, 128 - D - 4), jnp.float32)], axis=1)
    a_all = jnp.concatenate([a_z, a_q], axis=0)
    w_all = jnp.concatenate(
        [W1, b1[None, :], jnp.zeros((128 - D - 4, DOUT), jnp.float32)],
        axis=0)
    zq = _run_mm(a_all, w_all)
    z = zq[:B * NPB]
    q = zq[B * NPB:]

    x_out = _run_sc(pos[:, 0], pos[:, 1], pos[:, 2],
                    sel, cx, cy, cz, q, z)
    pos_out = pos_i
    batch_out = jnp.repeat(jnp.arange(B, dtype=jnp.int32), S)
    return (x_out, pos_out, batch_out)


# speedup vs baseline: 1.1481x; 1.1481x over previous
import functools

import jax
import jax.numpy as jnp
import numpy as np
from jax import lax
from jax.experimental import pallas as pl
from jax.experimental.pallas import tpu as pltpu
from jax.experimental.pallas import tpu_sc as plsc

B = 8
NPB = 4096
D = 64
S = 2048
K = 64
R2 = np.float32(0.2 * 0.2)
DOUT = 128

NW = 32
CPW = (B * S) // NW
IDXBUF = 80
BLK = 64


def _fps_body(px_ref, py_ref, pz_ref, sel_ref, cx_ref, cy_ref, cz_ref):
    px = px_ref[...]
    py = py_ref[...]
    pz = pz_ref[...]
    i32 = jnp.int32
    r = lax.broadcasted_iota(i32, (B, 32, 128), 1)
    l = lax.broadcasted_iota(i32, (B, 32, 128), 2)
    pt_iota = r * 128 + l
    rs = lax.broadcasted_iota(i32, (B, 16, 128), 1)
    ls = lax.broadcasted_iota(i32, (B, 16, 128), 2)
    sel_iota = rs * 128 + ls

    d0 = jnp.full((B, 32, 128), jnp.inf, jnp.float32)
    arg0 = jnp.zeros((B, 1, 1), i32)
    sel0 = jnp.zeros((B, 16, 128), i32)
    c0 = jnp.zeros((B, 16, 128), jnp.float32)

    def body(i, st):
        arg_prev, d, sel, cx, cy, cz = st
        onehot = pt_iota == arg_prev
        cxs = jnp.sum(jnp.where(onehot, px, 0.0), axis=(1, 2), keepdims=True)
        cys = jnp.sum(jnp.where(onehot, py, 0.0), axis=(1, 2), keepdims=True)
        czs = jnp.sum(jnp.where(onehot, pz, 0.0), axis=(1, 2), keepdims=True)
        wm = sel_iota == (i - 1)
        cx = jnp.where(wm, cxs, cx)
        cy = jnp.where(wm, cys, cy)
        cz = jnp.where(wm, czs, cz)
        dx = px - cxs
        dy = py - cys
        dz = pz - czs
        dd = dx * dx + dy * dy + dz * dz
        d = jnp.minimum(d, dd)
        maxd = jnp.max(d, axis=(1, 2), keepdims=True)
        cand = jnp.where(d == maxd, pt_iota, NPB)
        arg = jnp.min(cand, axis=(1, 2), keepdims=True)
        sel = jnp.where(sel_iota == i, arg, sel)
        return arg, d, sel, cx, cy, cz

    _, _, sel, cx, cy, cz = lax.fori_loop(
        1, S + 1, body, (arg0, d0, sel0, c0, c0, c0))
    sel_ref[...] = sel
    cx_ref[...] = cx
    cy_ref[...] = cy
    cz_ref[...] = cz


def _run_fps(px, py, pz):
    out_shapes = (
        jax.ShapeDtypeStruct((B, 16, 128), jnp.int32),
        jax.ShapeDtypeStruct((B, 16, 128), jnp.float32),
        jax.ShapeDtypeStruct((B, 16, 128), jnp.float32),
        jax.ShapeDtypeStruct((B, 16, 128), jnp.float32),
    )
    return pl.pallas_call(_fps_body, out_shape=out_shapes)(px, py, pz)


def _mm_body(a_ref, w_ref, o_ref):
    o_ref[...] = jnp.dot(a_ref[...], w_ref[...],
                         preferred_element_type=jnp.float32,
                         precision=lax.Precision.HIGHEST)


def _run_mm(a, w):
    n = a.shape[0]
    tile = 1024
    grid = (n // tile,)
    return pl.pallas_call(
        _mm_body,
        grid=grid,
        in_specs=[
            pl.BlockSpec((tile, 128), lambda i: (i, 0)),
            pl.BlockSpec((128, 128), lambda i: (0, 0)),
        ],
        out_specs=pl.BlockSpec((tile, 128), lambda i: (i, 0)),
        out_shape=jax.ShapeDtypeStruct((n, 128), jnp.float32),
    )(a, w)


def _sc_body(px_hbm, py_hbm, pz_hbm, sel_hbm, cx_hbm, cy_hbm, cz_hbm,
             q_hbm, z_hbm, out_hbm,
             px_v, py_v, pz_v, sel_v, cx_v, cy_v, cz_v,
             q_v, idx_v, zbuf_v, outbuf_v, sem, gsem):
    i32 = jnp.int32
    f32 = jnp.float32
    wid = lax.axis_index("s") * 2 + lax.axis_index("c")
    cloud = wid // 4
    cbase = wid * CPW
    gbase = cloud * NPB

    pltpu.sync_copy(px_hbm.at[pl.ds(gbase, NPB)], px_v)
    pltpu.sync_copy(py_hbm.at[pl.ds(gbase, NPB)], py_v)
    pltpu.sync_copy(pz_hbm.at[pl.ds(gbase, NPB)], pz_v)
    pltpu.sync_copy(sel_hbm.at[pl.ds(cbase, CPW)], sel_v)
    pltpu.sync_copy(cx_hbm.at[pl.ds(cbase, CPW)], cx_v)
    pltpu.sync_copy(cy_hbm.at[pl.ds(cbase, CPW)], cy_v)
    pltpu.sync_copy(cz_hbm.at[pl.ds(cbase, CPW)], cz_v)

    lane = lax.broadcasted_iota(i32, (16,), 0)
    neg_inf = jnp.full((16,), -jnp.inf, f32)

    def _splat(v, t):
        tv = jnp.full((16,), t, i32)
        scalar = jnp.sum(jnp.where(lane == tv, v, jnp.zeros((16,), v.dtype)),
                         axis=0)
        return jnp.full((16,), scalar, v.dtype)

    gbase_v = jnp.full((16,), gbase, i32)

    def selection(sw, p):
        g = sw // 16
        t = sw - g * 16
        selv = _splat(sel_v[pl.ds(g * 16, 16)], t)
        cxv = _splat(cx_v[pl.ds(g * 16, 16)], t)
        cyv = _splat(cy_v[pl.ds(g * 16, 16)], t)
        czv = _splat(cz_v[pl.ds(g * 16, 16)], t)
        safe = selv + gbase_v
        for tt in range(IDXBUF // 16):
            idx_v[p, pl.ds(tt * 16, 16)] = safe

        def cond(st):
            c, offs, _jidx, _off = st
            return (c < NPB // 16) & (offs < K)

        def chunk(st):
            c, offs, jidx, off = st
            pxc = px_v[pl.ds(c * 16, 16)]
            pyc = py_v[pl.ds(c * 16, 16)]
            pzc = pz_v[pl.ds(c * 16, 16)]
            dx = pxc - cxv
            dy = pyc - cyv
            dz = pzc - czv
            d2 = dx * dx + dy * dy + dz * dz
            m = (d2 <= R2) & (jidx != selv)
            incl = plsc.cumsum(m.astype(i32))
            posn = off + incl - 1
            plsc.store_scatter(idx_v.at[p], [posn], jidx + gbase_v, mask=m)
            cnt = plsc.all_reduce_population_count(m)
            return (c + 1, offs + jnp.max(cnt), jidx + 16, off + cnt)

        _, offs, _, _ = lax.while_loop(
            cond, chunk, (0, 0, lane, jnp.zeros((16,), i32)))
        return jnp.minimum(offs, K)

    def start_gather(p):
        return pltpu.async_copy(z_hbm.at[idx_v.at[p]], zbuf_v.at[p], gsem)

    def wait_gather(p):
        pltpu.make_async_copy(z_hbm.at[idx_v.at[p]], zbuf_v.at[p],
                              gsem).wait()

    def do_max(s, pm, nr):
        def row_body(rr, accs):
            return tuple(
                jnp.maximum(a, zbuf_v[pm, rr, pl.ds(kk * 16, 16)])
                for kk, a in enumerate(accs))

        accs = lax.fori_loop(0, nr, row_body, (neg_inf,) * 8)
        for kk in range(8):
            res = jnp.maximum(accs[kk] - q_v[s, pl.ds(kk * 16, 16)], 0.0)
            outbuf_v[s, pl.ds(kk * 16, 16)] = res

    def block_body(b, _):
        pltpu.sync_copy(q_hbm.at[pl.ds(cbase + b * BLK, BLK)], q_v)
        nr0 = selection(b * BLK, 0)
        start_gather(0)

        def pipe(s, nr_prev):
            p = s % 2
            pm = 1 - p
            nr = selection(b * BLK + s, p)
            wait_gather(pm)
            start_gather(p)
            do_max(s - 1, pm, nr_prev)
            return nr

        nr_last = lax.fori_loop(1, BLK, pipe, nr0)
        pl_last = (BLK - 1) % 2
        wait_gather(pl_last)
        do_max(BLK - 1, pl_last, nr_last)
        pltpu.sync_copy(outbuf_v, out_hbm.at[pl.ds(cbase + b * BLK, BLK)])
        return 0

    lax.fori_loop(0, CPW // BLK, block_body, 0)


def _run_sc(px, py, pz, sel, cx, cy, cz, q, z):
    mesh = plsc.VectorSubcoreMesh(core_axis_name="c", subcore_axis_name="s")
    kfn = pl.kernel(
        _sc_body,
        out_type=jax.ShapeDtypeStruct((B * S, DOUT), jnp.float32),
        mesh=mesh,
        compiler_params=pltpu.CompilerParams(needs_layout_passes=False),
        scratch_types=[
            pltpu.VMEM((NPB,), jnp.float32),
            pltpu.VMEM((NPB,), jnp.float32),
            pltpu.VMEM((NPB,), jnp.float32),
            pltpu.VMEM((CPW,), jnp.int32),
            pltpu.VMEM((CPW,), jnp.float32),
            pltpu.VMEM((CPW,), jnp.float32),
            pltpu.VMEM((CPW,), jnp.float32),
            pltpu.VMEM((BLK, DOUT), jnp.float32),
            pltpu.VMEM((2, IDXBUF), jnp.int32),
            pltpu.VMEM((2, IDXBUF, DOUT), jnp.float32),
            pltpu.VMEM((BLK, DOUT), jnp.float32),
            pltpu.SemaphoreType.DMA,
            pltpu.SemaphoreType.DMA,
        ],
    )
    return kfn(px, py, pz, sel, cx, cy, cz, q, z)


def kernel(x, pos, batch, W1, b1):
    posb = pos.reshape(B, NPB, 3)
    px = posb[:, :, 0].reshape(B, 32, 128)
    py = posb[:, :, 1].reshape(B, 32, 128)
    pz = posb[:, :, 2].reshape(B, 32, 128)

    sel3, cx3, cy3, cz3 = _run_fps(px, py, pz)
    sel = sel3.reshape(B * S)
    cx = cx3.reshape(B * S)
    cy = cy3.reshape(B * S)
    cz = cz3.reshape(B * S)
    pos_i = jnp.stack([cx, cy, cz], axis=-1)

    a_z = jnp.concatenate(
        [x, pos, jnp.zeros((B * NPB, 128 - D - 3), jnp.float32)], axis=1)
    a_q = jnp.concatenate(
        [jnp.zeros((B * S, D), jnp.float32), pos_i,
         jnp.full((B * S, 1), -1.0, jnp.float32),
         jnp.zeros((B * S, 128 - D - 4), jnp.float32)], axis=1)
    a_all = jnp.concatenate([a_z, a_q], axis=0)
    w_all = jnp.concatenate(
        [W1, b1[None, :], jnp.zeros((128 - D - 4, DOUT), jnp.float32)],
        axis=0)
    zq = _run_mm(a_all, w_all)
    z = zq[:B * NPB]
    q = zq[B * NPB:]

    x_out = _run_sc(pos[:, 0], pos[:, 1], pos[:, 2],
                    sel, cx, cy, cz, q, z)
    pos_out = pos_i
    batch_out = jnp.repeat(jnp.arange(B, dtype=jnp.int32), S)
    return (x_out, pos_out, batch_out)


# speedup vs baseline: 1.4373x; 1.2518x over previous
import functools

import jax
import jax.numpy as jnp
import numpy as np
from jax import lax
from jax.experimental import pallas as pl
from jax.experimental.pallas import tpu as pltpu
from jax.experimental.pallas import tpu_sc as plsc

B = 8
NPB = 4096
D = 64
S = 2048
K = 64
R2 = np.float32(0.2 * 0.2)
DOUT = 128

NW = 32
CPW = (B * S) // NW
IDXBUF = 128
BLK = 64


def _fps_body(px_ref, py_ref, pz_ref, sel_ref, cx_ref, cy_ref, cz_ref):
    px = px_ref[...]
    py = py_ref[...]
    pz = pz_ref[...]
    i32 = jnp.int32
    r = lax.broadcasted_iota(i32, (B, 32, 128), 1)
    l = lax.broadcasted_iota(i32, (B, 32, 128), 2)
    pt_iota = r * 128 + l
    rs = lax.broadcasted_iota(i32, (B, 16, 128), 1)
    ls = lax.broadcasted_iota(i32, (B, 16, 128), 2)
    sel_iota = rs * 128 + ls

    d0 = jnp.full((B, 32, 128), jnp.inf, jnp.float32)
    arg0 = jnp.zeros((B, 1, 1), i32)
    sel0 = jnp.zeros((B, 16, 128), i32)
    c0 = jnp.zeros((B, 16, 128), jnp.float32)

    def body(i, st):
        arg_prev, d, sel, cx, cy, cz = st
        onehot = pt_iota == arg_prev
        cxs = jnp.sum(jnp.where(onehot, px, 0.0), axis=(1, 2), keepdims=True)
        cys = jnp.sum(jnp.where(onehot, py, 0.0), axis=(1, 2), keepdims=True)
        czs = jnp.sum(jnp.where(onehot, pz, 0.0), axis=(1, 2), keepdims=True)
        wm = sel_iota == (i - 1)
        cx = jnp.where(wm, cxs, cx)
        cy = jnp.where(wm, cys, cy)
        cz = jnp.where(wm, czs, cz)
        dx = px - cxs
        dy = py - cys
        dz = pz - czs
        dd = dx * dx + dy * dy + dz * dz
        d = jnp.minimum(d, dd)
        maxd = jnp.max(d, axis=(1, 2), keepdims=True)
        cand = jnp.where(d == maxd, pt_iota, NPB)
        arg = jnp.min(cand, axis=(1, 2), keepdims=True)
        sel = jnp.where(sel_iota == i, arg, sel)
        return arg, d, sel, cx, cy, cz

    _, _, sel, cx, cy, cz = lax.fori_loop(
        1, S + 1, body, (arg0, d0, sel0, c0, c0, c0))
    sel_ref[...] = sel
    cx_ref[...] = cx
    cy_ref[...] = cy
    cz_ref[...] = cz


def _run_fps(px, py, pz):
    out_shapes = (
        jax.ShapeDtypeStruct((B, 16, 128), jnp.int32),
        jax.ShapeDtypeStruct((B, 16, 128), jnp.float32),
        jax.ShapeDtypeStruct((B, 16, 128), jnp.float32),
        jax.ShapeDtypeStruct((B, 16, 128), jnp.float32),
    )
    return pl.pallas_call(_fps_body, out_shape=out_shapes)(px, py, pz)


def _mm_body(a_ref, w_ref, o_ref):
    o_ref[...] = jnp.dot(a_ref[...], w_ref[...],
                         preferred_element_type=jnp.float32,
                         precision=lax.Precision.HIGHEST)


def _run_mm(a, w):
    n = a.shape[0]
    tile = 1024
    grid = (n // tile,)
    return pl.pallas_call(
        _mm_body,
        grid=grid,
        in_specs=[
            pl.BlockSpec((tile, 128), lambda i: (i, 0)),
            pl.BlockSpec((128, 128), lambda i: (0, 0)),
        ],
        out_specs=pl.BlockSpec((tile, 128), lambda i: (i, 0)),
        out_shape=jax.ShapeDtypeStruct((n, 128), jnp.float32),
    )(a, w)


def _sc_body(px_hbm, py_hbm, pz_hbm, sel_hbm, cx_hbm, cy_hbm, cz_hbm,
             q_hbm, z_hbm, out_hbm,
             px_v, py_v, pz_v, sel_v, cx_v, cy_v, cz_v,
             q_v, idx_v, zbuf_v, outbuf_v, sem, gsem):
    i32 = jnp.int32
    f32 = jnp.float32
    wid = lax.axis_index("s") * 2 + lax.axis_index("c")
    cloud = wid // 4
    cbase = wid * CPW
    gbase = cloud * NPB

    pltpu.sync_copy(px_hbm.at[pl.ds(gbase, NPB)], px_v)
    pltpu.sync_copy(py_hbm.at[pl.ds(gbase, NPB)], py_v)
    pltpu.sync_copy(pz_hbm.at[pl.ds(gbase, NPB)], pz_v)
    pltpu.sync_copy(sel_hbm.at[pl.ds(cbase, CPW)], sel_v)
    pltpu.sync_copy(cx_hbm.at[pl.ds(cbase, CPW)], cx_v)
    pltpu.sync_copy(cy_hbm.at[pl.ds(cbase, CPW)], cy_v)
    pltpu.sync_copy(cz_hbm.at[pl.ds(cbase, CPW)], cz_v)

    lane = lax.broadcasted_iota(i32, (16,), 0)
    neg_inf = jnp.full((16,), -jnp.inf, f32)

    def _splat(v, t):
        tv = jnp.full((16,), t, i32)
        scalar = jnp.sum(jnp.where(lane == tv, v, jnp.zeros((16,), v.dtype)),
                         axis=0)
        return jnp.full((16,), scalar, v.dtype)

    gbase_v = jnp.full((16,), gbase, i32)

    def selection(sw, p):
        g = sw // 16
        t = sw - g * 16
        selv = _splat(sel_v[pl.ds(g * 16, 16)], t)
        cxv = _splat(cx_v[pl.ds(g * 16, 16)], t)
        cyv = _splat(cy_v[pl.ds(g * 16, 16)], t)
        czv = _splat(cz_v[pl.ds(g * 16, 16)], t)
        safe = selv + gbase_v
        for tt in range(IDXBUF // 16):
            idx_v[p, pl.ds(tt * 16, 16)] = safe

        def cond(st):
            gc, offs, _off = st
            return (gc < NPB // 64) & (offs < K)

        def group(st):
            gc, offs, off = st
            base = gc * 64
            for u in range(4):
                pxc = px_v[pl.ds(base + u * 16, 16)]
                pyc = py_v[pl.ds(base + u * 16, 16)]
                pzc = pz_v[pl.ds(base + u * 16, 16)]
                jidx = jnp.full((16,), base + u * 16, i32) + lane
                dx = pxc - cxv
                dy = pyc - cyv
                dz = pzc - czv
                d2 = dx * dx + dy * dy + dz * dz
                m = (d2 <= R2) & (jidx != selv)
                incl = plsc.cumsum(m.astype(i32))
                posn = off + incl - 1
                plsc.store_scatter(idx_v.at[p], [posn], jidx + gbase_v,
                                   mask=m)
                off = off + plsc.all_reduce_population_count(m)
            return (gc + 1, jnp.max(off), off)

        _, offs, _ = lax.while_loop(
            cond, group, (0, 0, jnp.zeros((16,), i32)))
        return jnp.minimum(offs, K)

    def start_gather(p):
        return pltpu.async_copy(z_hbm.at[idx_v.at[p, pl.ds(0, K)]],
                                zbuf_v.at[p], gsem)

    def wait_gather(p):
        pltpu.make_async_copy(z_hbm.at[idx_v.at[p, pl.ds(0, K)]],
                              zbuf_v.at[p], gsem).wait()

    def do_max(s, pm, nr):
        def row_body(rr, accs):
            return tuple(
                jnp.maximum(a, zbuf_v[pm, rr, pl.ds(kk * 16, 16)])
                for kk, a in enumerate(accs))

        accs = lax.fori_loop(0, nr, row_body, (neg_inf,) * 8)
        for kk in range(8):
            res = jnp.maximum(accs[kk] - q_v[s, pl.ds(kk * 16, 16)], 0.0)
            outbuf_v[s, pl.ds(kk * 16, 16)] = res

    def block_body(b, _):
        pltpu.sync_copy(q_hbm.at[pl.ds(cbase + b * BLK, BLK)], q_v)
        nr0 = selection(b * BLK, 0)
        start_gather(0)

        def pipe(s, nr_prev):
            p = s % 2
            pm = 1 - p
            nr = selection(b * BLK + s, p)
            wait_gather(pm)
            start_gather(p)
            do_max(s - 1, pm, nr_prev)
            return nr

        nr_last = lax.fori_loop(1, BLK, pipe, nr0)
        pl_last = (BLK - 1) % 2
        wait_gather(pl_last)
        do_max(BLK - 1, pl_last, nr_last)
        pltpu.sync_copy(outbuf_v, out_hbm.at[pl.ds(cbase + b * BLK, BLK)])
        return 0

    lax.fori_loop(0, CPW // BLK, block_body, 0)


def _run_sc(px, py, pz, sel, cx, cy, cz, q, z):
    mesh = plsc.VectorSubcoreMesh(core_axis_name="c", subcore_axis_name="s")
    kfn = pl.kernel(
        _sc_body,
        out_type=jax.ShapeDtypeStruct((B * S, DOUT), jnp.float32),
        mesh=mesh,
        compiler_params=pltpu.CompilerParams(needs_layout_passes=False),
        scratch_types=[
            pltpu.VMEM((NPB,), jnp.float32),
            pltpu.VMEM((NPB,), jnp.float32),
            pltpu.VMEM((NPB,), jnp.float32),
            pltpu.VMEM((CPW,), jnp.int32),
            pltpu.VMEM((CPW,), jnp.float32),
            pltpu.VMEM((CPW,), jnp.float32),
            pltpu.VMEM((CPW,), jnp.float32),
            pltpu.VMEM((BLK, DOUT), jnp.float32),
            pltpu.VMEM((2, IDXBUF), jnp.int32),
            pltpu.VMEM((2, K, DOUT), jnp.float32),
            pltpu.VMEM((BLK, DOUT), jnp.float32),
            pltpu.SemaphoreType.DMA,
            pltpu.SemaphoreType.DMA,
        ],
    )
    return kfn(px, py, pz, sel, cx, cy, cz, q, z)


def kernel(x, pos, batch, W1, b1):
    posb = pos.reshape(B, NPB, 3)
    px = posb[:, :, 0].reshape(B, 32, 128)
    py = posb[:, :, 1].reshape(B, 32, 128)
    pz = posb[:, :, 2].reshape(B, 32, 128)

    sel3, cx3, cy3, cz3 = _run_fps(px, py, pz)
    sel = sel3.reshape(B * S)
    cx = cx3.reshape(B * S)
    cy = cy3.reshape(B * S)
    cz = cz3.reshape(B * S)
    pos_i = jnp.stack([cx, cy, cz], axis=-1)

    a_z = jnp.concatenate(
        [x, pos, jnp.zeros((B * NPB, 128 - D - 3), jnp.float32)], axis=1)
    a_q = jnp.concatenate(
        [jnp.zeros((B * S, D), jnp.float32), pos_i,
         jnp.full((B * S, 1), -1.0, jnp.float32),
         jnp.zeros((B * S, 128 - D - 4), jnp.float32)], axis=1)
    a_all = jnp.concatenate([a_z, a_q], axis=0)
    w_all = jnp.concatenate(
        [W1, b1[None, :], jnp.zeros((128 - D - 4, DOUT), jnp.float32)],
        axis=0)
    zq = _run_mm(a_all, w_all)
    z = zq[:B * NPB]
    q = zq[B * NPB:]

    x_out = _run_sc(pos[:, 0], pos[:, 1], pos[:, 2],
                    sel, cx, cy, cz, q, z)
    pos_out = pos_i
    batch_out = jnp.repeat(jnp.arange(B, dtype=jnp.int32), S)
    return (x_out, pos_out, batch_out)


# speedup vs baseline: 1.4382x; 1.0006x over previous
import functools

import jax
import jax.numpy as jnp
import numpy as np
from jax import lax
from jax.experimental import pallas as pl
from jax.experimental.pallas import tpu as pltpu
from jax.experimental.pallas import tpu_sc as plsc

B = 8
NPB = 4096
D = 64
S = 2048
K = 64
R2 = np.float32(0.2 * 0.2)
DOUT = 128

NW = 32
CPW = (B * S) // NW
IDXBUF = 128
BLK = 64


def _fps_body(px_ref, py_ref, pz_ref, sel_ref, cx_ref, cy_ref, cz_ref):
    px = px_ref[...]
    py = py_ref[...]
    pz = pz_ref[...]
    i32 = jnp.int32
    r = lax.broadcasted_iota(i32, (B, 32, 128), 1)
    l = lax.broadcasted_iota(i32, (B, 32, 128), 2)
    pt_iota = r * 128 + l
    rs = lax.broadcasted_iota(i32, (B, 16, 128), 1)
    ls = lax.broadcasted_iota(i32, (B, 16, 128), 2)
    sel_iota = rs * 128 + ls

    d0 = jnp.full((B, 32, 128), jnp.inf, jnp.float32)
    arg0 = jnp.zeros((B, 1, 1), i32)
    sel0 = jnp.zeros((B, 16, 128), i32)
    c0 = jnp.zeros((B, 16, 128), jnp.float32)

    def body(i, st):
        arg_prev, d, sel, cx, cy, cz = st
        onehot = pt_iota == arg_prev
        cxs = jnp.sum(jnp.where(onehot, px, 0.0), axis=(1, 2), keepdims=True)
        cys = jnp.sum(jnp.where(onehot, py, 0.0), axis=(1, 2), keepdims=True)
        czs = jnp.sum(jnp.where(onehot, pz, 0.0), axis=(1, 2), keepdims=True)
        wm = sel_iota == (i - 1)
        cx = jnp.where(wm, cxs, cx)
        cy = jnp.where(wm, cys, cy)
        cz = jnp.where(wm, czs, cz)
        dx = px - cxs
        dy = py - cys
        dz = pz - czs
        dd = dx * dx + dy * dy + dz * dz
        d = jnp.minimum(d, dd)
        maxd = jnp.max(d, axis=(1, 2), keepdims=True)
        cand = jnp.where(d == maxd, pt_iota, NPB)
        arg = jnp.min(cand, axis=(1, 2), keepdims=True)
        sel = jnp.where(sel_iota == i, arg, sel)
        return arg, d, sel, cx, cy, cz

    _, _, sel, cx, cy, cz = lax.fori_loop(
        1, S + 1, body, (arg0, d0, sel0, c0, c0, c0))
    sel_ref[...] = sel
    cx_ref[...] = cx
    cy_ref[...] = cy
    cz_ref[...] = cz


def _run_fps(px, py, pz):
    out_shapes = (
        jax.ShapeDtypeStruct((B, 16, 128), jnp.int32),
        jax.ShapeDtypeStruct((B, 16, 128), jnp.float32),
        jax.ShapeDtypeStruct((B, 16, 128), jnp.float32),
        jax.ShapeDtypeStruct((B, 16, 128), jnp.float32),
    )
    return pl.pallas_call(_fps_body, out_shape=out_shapes)(px, py, pz)


def _mm_body(a_ref, w_ref, o_ref):
    o_ref[...] = jnp.dot(a_ref[...], w_ref[...],
                         preferred_element_type=jnp.float32,
                         precision=lax.Precision.HIGHEST)


def _run_mm(a, w):
    n = a.shape[0]
    tile = 1024
    grid = (n // tile,)
    return pl.pallas_call(
        _mm_body,
        grid=grid,
        in_specs=[
            pl.BlockSpec((tile, 128), lambda i: (i, 0)),
            pl.BlockSpec((128, 128), lambda i: (0, 0)),
        ],
        out_specs=pl.BlockSpec((tile, 128), lambda i: (i, 0)),
        out_shape=jax.ShapeDtypeStruct((n, 128), jnp.float32),
    )(a, w)


def _sc_body(px_hbm, py_hbm, pz_hbm, sel_hbm, cx_hbm, cy_hbm, cz_hbm,
             q_hbm, z_hbm, out_hbm,
             px_v, py_v, pz_v, sel_v, cx_v, cy_v, cz_v,
             q_v, idx_v, zbuf_v, outbuf_v, sem, gsem):
    i32 = jnp.int32
    f32 = jnp.float32
    wid = lax.axis_index("s") * 2 + lax.axis_index("c")
    cloud = wid // 4
    cbase = wid * CPW
    gbase = cloud * NPB

    pltpu.sync_copy(px_hbm.at[pl.ds(gbase, NPB)], px_v)
    pltpu.sync_copy(py_hbm.at[pl.ds(gbase, NPB)], py_v)
    pltpu.sync_copy(pz_hbm.at[pl.ds(gbase, NPB)], pz_v)
    pltpu.sync_copy(sel_hbm.at[pl.ds(cbase, CPW)], sel_v)
    pltpu.sync_copy(cx_hbm.at[pl.ds(cbase, CPW)], cx_v)
    pltpu.sync_copy(cy_hbm.at[pl.ds(cbase, CPW)], cy_v)
    pltpu.sync_copy(cz_hbm.at[pl.ds(cbase, CPW)], cz_v)

    lane = lax.broadcasted_iota(i32, (16,), 0)
    neg_inf = jnp.full((16,), -jnp.inf, f32)

    def _splat(v, t):
        tv = jnp.full((16,), t, i32)
        scalar = jnp.sum(jnp.where(lane == tv, v, jnp.zeros((16,), v.dtype)),
                         axis=0)
        return jnp.full((16,), scalar, v.dtype)

    gbase_v = jnp.full((16,), gbase, i32)

    def selection(sw, p):
        g = sw // 16
        t = sw - g * 16
        selv = _splat(sel_v[pl.ds(g * 16, 16)], t)
        cxv = _splat(cx_v[pl.ds(g * 16, 16)], t)
        cyv = _splat(cy_v[pl.ds(g * 16, 16)], t)
        czv = _splat(cz_v[pl.ds(g * 16, 16)], t)
        safe = selv + gbase_v
        for tt in range(IDXBUF // 16):
            idx_v[p, pl.ds(tt * 16, 16)] = safe

        def cond(st):
            gc, offs, _off = st
            return (gc < NPB // 64) & (offs < K)

        def group(st):
            gc, offs, off = st
            base = gc * 64
            for u in range(4):
                pxc = px_v[pl.ds(base + u * 16, 16)]
                pyc = py_v[pl.ds(base + u * 16, 16)]
                pzc = pz_v[pl.ds(base + u * 16, 16)]
                jidxg = jnp.full((16,), gbase + base + u * 16, i32) + lane
                dx = pxc - cxv
                dy = pyc - cyv
                dz = pzc - czv
                d2 = dx * dx + dy * dy + dz * dz
                m = (d2 <= R2) & (jidxg != safe)
                incl = plsc.cumsum(m.astype(i32))
                posn = off + incl - 1
                plsc.store_scatter(idx_v.at[p], [posn], jidxg, mask=m)
                off = off + plsc.all_reduce_population_count(m)
            return (gc + 1, jnp.max(off), off)

        _, offs, _ = lax.while_loop(
            cond, group, (0, 0, jnp.zeros((16,), i32)))
        return jnp.minimum(offs, K)

    def start_gather(p):
        return pltpu.async_copy(z_hbm.at[idx_v.at[p, pl.ds(0, K)]],
                                zbuf_v.at[p], gsem)

    def wait_gather(p):
        pltpu.make_async_copy(z_hbm.at[idx_v.at[p, pl.ds(0, K)]],
                              zbuf_v.at[p], gsem).wait()

    def do_max(s, pm, nr):
        def row_body(rr, accs):
            return tuple(
                jnp.maximum(a, zbuf_v[pm, rr, pl.ds(kk * 16, 16)])
                for kk, a in enumerate(accs))

        accs = lax.fori_loop(0, nr, row_body, (neg_inf,) * 8)
        for kk in range(8):
            res = jnp.maximum(accs[kk] - q_v[s, pl.ds(kk * 16, 16)], 0.0)
            outbuf_v[s, pl.ds(kk * 16, 16)] = res

    def block_body(b, _):
        pltpu.sync_copy(q_hbm.at[pl.ds(cbase + b * BLK, BLK)], q_v)
        nr0 = selection(b * BLK, 0)
        start_gather(0)

        def pipe(s, nr_prev):
            p = s % 2
            pm = 1 - p
            nr = selection(b * BLK + s, p)
            wait_gather(pm)
            start_gather(p)
            do_max(s - 1, pm, nr_prev)
            return nr

        nr_last = lax.fori_loop(1, BLK, pipe, nr0)
        pl_last = (BLK - 1) % 2
        wait_gather(pl_last)
        do_max(BLK - 1, pl_last, nr_last)
        pltpu.sync_copy(outbuf_v, out_hbm.at[pl.ds(cbase + b * BLK, BLK)])
        return 0

    lax.fori_loop(0, CPW // BLK, block_body, 0)


def _run_sc(px, py, pz, sel, cx, cy, cz, q, z):
    mesh = plsc.VectorSubcoreMesh(core_axis_name="c", subcore_axis_name="s")
    kfn = pl.kernel(
        _sc_body,
        out_type=jax.ShapeDtypeStruct((B * S, DOUT), jnp.float32),
        mesh=mesh,
        compiler_params=pltpu.CompilerParams(needs_layout_passes=False),
        scratch_types=[
            pltpu.VMEM((NPB,), jnp.float32),
            pltpu.VMEM((NPB,), jnp.float32),
            pltpu.VMEM((NPB,), jnp.float32),
            pltpu.VMEM((CPW,), jnp.int32),
            pltpu.VMEM((CPW,), jnp.float32),
            pltpu.VMEM((CPW,), jnp.float32),
            pltpu.VMEM((CPW,), jnp.float32),
            pltpu.VMEM((BLK, DOUT), jnp.float32),
            pltpu.VMEM((2, IDXBUF), jnp.int32),
            pltpu.VMEM((2, K, DOUT), jnp.float32),
            pltpu.VMEM((BLK, DOUT), jnp.float32),
            pltpu.SemaphoreType.DMA,
            pltpu.SemaphoreType.DMA,
        ],
    )
    return kfn(px, py, pz, sel, cx, cy, cz, q, z)


def kernel(x, pos, batch, W1, b1):
    posb = pos.reshape(B, NPB, 3)
    px = posb[:, :, 0].reshape(B, 32, 128)
    py = posb[:, :, 1].reshape(B, 32, 128)
    pz = posb[:, :, 2].reshape(B, 32, 128)

    sel3, cx3, cy3, cz3 = _run_fps(px, py, pz)
    sel = sel3.reshape(B * S)
    cx = cx3.reshape(B * S)
    cy = cy3.reshape(B * S)
    cz = cz3.reshape(B * S)
    pos_i = jnp.stack([cx, cy, cz], axis=-1)

    a_z = jnp.concatenate(
        [x, pos, jnp.zeros((B * NPB, 128 - D - 3), jnp.float32)], axis=1)
    a_q = jnp.concatenate(
        [jnp.zeros((B * S, D), jnp.float32), pos_i,
         jnp.full((B * S, 1), -1.0, jnp.float32),
         jnp.zeros((B * S, 128 - D - 4), jnp.float32)], axis=1)
    a_all = jnp.concatenate([a_z, a_q], axis=0)
    w_all = jnp.concatenate(
        [W1, b1[None, :], jnp.zeros((128 - D - 4, DOUT), jnp.float32)],
        axis=0)
    zq = _run_mm(a_all, w_all)
    z = zq[:B * NPB]
    q = zq[B * NPB:]

    x_out = _run_sc(pos[:, 0], pos[:, 1], pos[:, 2],
                    sel, cx, cy, cz,
                    q, z)
    pos_out = pos_i
    batch_out = jnp.repeat(jnp.arange(B, dtype=jnp.int32), S)
    return (x_out, pos_out, batch_out)
